# baseline (speedup 1.0000x reference)
import functools

import jax
import jax.numpy as jnp
from jax import lax
from jax.experimental import pallas as pl
from jax.experimental.pallas import tpu as pltpu
from jax.experimental.pallas import tpu_sc as plsc

_GATHER_WINDOW = 128


def _sc_gather(table, flat_idx):
    n = flat_idx.shape[0]
    d = table.shape[1]
    mesh = plsc.VectorSubcoreMesh(core_axis_name="c", subcore_axis_name="s")

    @functools.partial(
        pl.kernel,
        out_type=jax.ShapeDtypeStruct((n, d), jnp.float32),
        mesh=mesh,
        compiler_params=pltpu.CompilerParams(use_tc_tiling_on_sc=False),
    )
    def gk(table_hbm, idx_hbm, out_hbm):
        def body(i_vmem, o_vmem):
            pltpu.sync_copy(table_hbm.at[i_vmem.at[0]], o_vmem)

        pltpu.emit_pipeline(
            body,
            grid=(n // _GATHER_WINDOW,),
            in_specs=[
                pl.BlockSpec((1, _GATHER_WINDOW), index_map=lambda i: (0, i))
            ],
            out_specs=[
                pl.BlockSpec((_GATHER_WINDOW, d), index_map=lambda i: (i, 0))
            ],
            core_axis_name=("c", "s"),
            dimension_semantics=(pltpu.PARALLEL,),
        )(idx_hbm, out_hbm)

    return gk(table, flat_idx.reshape(1, n))


def _tc_finish(gathered_f, numeric, cat_i, lin_w, lin_b2):
    f, b, d = gathered_f.shape
    n_num = numeric.shape[1]
    bb = 256
    g = b // bb

    def body(g_ref, num_ref, cat_ref, w_ref, b_ref, o_ref):
        v = g_ref[...]
        s = jnp.sum(v, axis=0)
        ss = jnp.sum(v * v, axis=0)
        order2 = 0.5 * jnp.sum(s * s - ss, axis=-1)
        w = w_ref[...]
        catf = cat_ref[...].astype(jnp.float32)
        order1 = (
            jnp.sum(num_ref[...] * w[:, :n_num], axis=1)
            + jnp.sum(catf * w[:, n_num:], axis=1)
            + b_ref[0, 0]
        )
        o_ref[...] = jax.nn.sigmoid(order1 + order2).reshape(1, 1, bb)

    return pl.pallas_call(
        body,
        grid=(g,),
        in_specs=[
            pl.BlockSpec((f, bb, d), lambda i: (0, i, 0)),
            pl.BlockSpec((bb, n_num), lambda i: (i, 0)),
            pl.BlockSpec((bb, f), lambda i: (i, 0)),
            pl.BlockSpec((1, n_num + f), lambda i: (0, 0)),
            pl.BlockSpec((1, 1), lambda i: (0, 0)),
        ],
        out_specs=pl.BlockSpec((1, 1, bb), lambda i: (i, 0, 0)),
        out_shape=jax.ShapeDtypeStruct((g, 1, bb), jnp.float32),
    )(gathered_f, numeric, cat_i, lin_w, lin_b2)


def kernel(numeric_features, cat_features, lin_w, lin_b, emb_tables):
    b, f = cat_features.shape
    n_fields, vocab, d = emb_tables.shape
    cat_i = cat_features.astype(jnp.int32)
    flat_idx = (
        cat_i.T + jnp.arange(f, dtype=jnp.int32)[:, None] * vocab
    ).reshape(-1)
    table = emb_tables.reshape(n_fields * vocab, d)
    gathered = _sc_gather(table, flat_idx)
    out = _tc_finish(
        gathered.reshape(f, b, d),
        numeric_features,
        cat_i,
        lin_w,
        lin_b.reshape(1, 1),
    )
    return out.reshape(b)


# speedup vs baseline: 1.7933x; 1.7933x over previous
import functools

import jax
import jax.numpy as jnp
from jax import lax
from jax.experimental import pallas as pl
from jax.experimental.pallas import tpu as pltpu
from jax.experimental.pallas import tpu_sc as plsc

_LANES = 16
_NWORKERS = 32


def _tc_pack_transpose(table_t):
    f, d, v = table_t.shape
    ng = f // 2
    vblk = 2560

    def body(x0_ref, x1_ref, o_ref):
        x0 = x0_ref[0]
        x1 = x1_ref[0]
        eye = jnp.eye(d, dtype=jnp.float32)
        x0t = lax.dot_general(x0, eye, (((0,), (0,)), ((), ())),
                              preferred_element_type=jnp.float32)
        x1t = lax.dot_general(x1, eye, (((0,), (0,)), ((), ())),
                              preferred_element_type=jnp.float32)
        o_ref[0] = jnp.concatenate([x0t, x1t], axis=1)

    return pl.pallas_call(
        body,
        grid=(ng, pl.cdiv(v, vblk)),
        in_specs=[
            pl.BlockSpec((1, d, vblk), lambda g, j: (2 * g, 0, j)),
            pl.BlockSpec((1, d, vblk), lambda g, j: (2 * g + 1, 0, j)),
        ],
        out_specs=pl.BlockSpec((1, vblk, 2 * d), lambda g, j: (g, j, 0)),
        out_shape=jax.ShapeDtypeStruct((ng, v, 2 * d), jnp.float32),
    )(table_t, table_t)


def _sc_fm_gather(tpack, cat_t):
    ng, v, d2 = tpack.shape
    d = d2 // 2
    f = 2 * ng
    n_b = cat_t.shape[1]
    bt = n_b // _NWORKERS

    mesh = plsc.VectorSubcoreMesh(core_axis_name="c", subcore_axis_name="s")

    @functools.partial(
        pl.kernel,
        out_type=jax.ShapeDtypeStruct((2, n_b, d), jnp.float32),
        mesh=mesh,
        scratch_types=[
            pltpu.VMEM((f, bt), jnp.int32),
            pltpu.VMEM((bt, d2), jnp.float32),
            pltpu.VMEM((bt, d2), jnp.float32),
            pltpu.VMEM((bt, d), jnp.float32),
            pltpu.VMEM((bt, d), jnp.float32),
            pltpu.SemaphoreType.DMA,
            pltpu.SemaphoreType.DMA,
        ],
        compiler_params=pltpu.CompilerParams(use_tc_tiling_on_sc=True),
    )
    def fk(tpack_hbm, cat_hbm, out_hbm, idx_v, bufa, bufb, s_t, ss_t,
           sema, semb):
        cid = lax.axis_index("c")
        sid = lax.axis_index("s")
        wid = sid * 2 + cid
        bbase = wid * bt

        pltpu.sync_copy(cat_hbm.at[:, pl.ds(bbase, bt)], idx_v)

        zero = jnp.zeros((_LANES,), jnp.float32)

        @pl.loop(0, bt)
        def _(r):
            for i in range(d // _LANES):
                s_t[r, pl.ds(i * _LANES, _LANES)] = zero
                ss_t[r, pl.ds(i * _LANES, _LANES)] = zero

        bufs = (bufa, bufb)
        sems = (sema, semb)

        def gather_q(q, buf, sem):
            return pltpu.async_copy(
                tpack_hbm.at[q // 2].at[idx_v.at[q]], buf, sem)

        @pl.when(True)
        def _():
            gather_q(0, bufa, sema)

        @pl.loop(0, f)
        def _(q):
            parity = lax.rem(q, 2)
            off = lax.rem(q, 2) * d
            for p in range(2):
                @pl.when(parity == p)
                def _():
                    pltpu.make_async_copy(
                        tpack_hbm.at[q // 2].at[idx_v.at[q]],
                        bufs[p], sems[p]).wait()

                    @pl.when(q + 1 < f)
                    def _():
                        gather_q(q + 1, bufs[1 - p], sems[1 - p])

                    buf = bufs[p]

                    @pl.loop(0, bt)
                    def _(r):
                        for c in range(d // _LANES):
                            x = buf[r, pl.ds(off + c * _LANES, _LANES)]
                            plsc.addupdate(
                                s_t.at[r, pl.ds(c * _LANES, _LANES)], x)
                            plsc.addupdate(
                                ss_t.at[r, pl.ds(c * _LANES, _LANES)],
                                x * x)

        pltpu.sync_copy(s_t, out_hbm.at[0, pl.ds(bbase, bt), :])
        pltpu.sync_copy(ss_t, out_hbm.at[1, pl.ds(bbase, bt), :])

    return fk(tpack, cat_t)


def _tc_finish(partials, numeric_t, cat_t, w_num_t, w_cat_t, lin_b2):
    _, n_b, d = partials.shape
    n_num = numeric_t.shape[0]
    f = cat_t.shape[0]
    bb = 512
    g = n_b // bb

    def body(p_ref, num_ref, cat_ref, wn_ref, wc_ref, b_ref, o_ref):
        p = p_ref[...]
        s = p[0]
        ss = p[1]
        order2 = 0.5 * jnp.sum(s * s - ss, axis=1)
        catf = cat_ref[...].astype(jnp.float32)
        order1 = (
            jnp.sum(num_ref[...] * wn_ref[...], axis=0)
            + jnp.sum(catf * wc_ref[...], axis=0)
            + b_ref[0, 0]
        )
        o_ref[...] = jax.nn.sigmoid(order1 + order2).reshape(1, 1, bb)

    return pl.pallas_call(
        body,
        grid=(g,),
        in_specs=[
            pl.BlockSpec((2, bb, d), lambda i: (0, i, 0)),
            pl.BlockSpec((n_num, bb), lambda i: (0, i)),
            pl.BlockSpec((f, bb), lambda i: (0, i)),
            pl.BlockSpec((n_num, 1), lambda i: (0, 0)),
            pl.BlockSpec((f, 1), lambda i: (0, 0)),
            pl.BlockSpec((1, 1), lambda i: (0, 0)),
        ],
        out_specs=pl.BlockSpec((1, 1, bb), lambda i: (i, 0, 0)),
        out_shape=jax.ShapeDtypeStruct((g, 1, bb), jnp.float32),
    )(partials, numeric_t, cat_t, w_num_t, w_cat_t, lin_b2)


def kernel(numeric_features, cat_features, lin_w, lin_b, emb_tables):
    b, f = cat_features.shape
    n_num = numeric_features.shape[1]
    cat_i = cat_features.astype(jnp.int32)
    table_t = jnp.transpose(emb_tables, (0, 2, 1))
    cat_t = jnp.transpose(cat_i, (1, 0))
    numeric_t = jnp.transpose(numeric_features, (1, 0))
    tpack = _tc_pack_transpose(table_t)
    partials = _sc_fm_gather(tpack, cat_t)
    out = _tc_finish(
        partials,
        numeric_t,
        cat_t,
        lin_w[:, :n_num].reshape(n_num, 1),
        lin_w[:, n_num:].reshape(f, 1),
        lin_b.reshape(1, 1),
    )
    return out.reshape(b)


# speedup vs baseline: 2.1632x; 1.2062x over previous
import functools

import jax
import jax.numpy as jnp
from jax import lax
from jax.experimental import pallas as pl
from jax.experimental.pallas import tpu as pltpu
from jax.experimental.pallas import tpu_sc as plsc

_LANES = 16
_NWORKERS = 32


def _tc_pack_transpose(table_t):
    f, d, v = table_t.shape
    ng = f // 2
    vblk = 2560

    def body(x_ref, o_ref):
        x = x_ref[...].reshape(2 * d, vblk)
        o_ref[0] = jnp.transpose(x, (1, 0))

    return pl.pallas_call(
        body,
        grid=(ng, pl.cdiv(v, vblk)),
        in_specs=[
            pl.BlockSpec((2, d, vblk), lambda g, j: (g, 0, j)),
        ],
        out_specs=pl.BlockSpec((1, vblk, 2 * d), lambda g, j: (g, j, 0)),
        out_shape=jax.ShapeDtypeStruct((ng, v, 2 * d), jnp.float32),
    )(table_t)


def _sc_fm_gather(tpack, cat_t):
    ng, v, d2 = tpack.shape
    d = d2 // 2
    f = 2 * ng
    n_b = cat_t.shape[1]
    bt = n_b // _NWORKERS

    mesh = plsc.VectorSubcoreMesh(core_axis_name="c", subcore_axis_name="s")

    @functools.partial(
        pl.kernel,
        out_type=jax.ShapeDtypeStruct((2, n_b, d), jnp.float32),
        mesh=mesh,
        scratch_types=[
            pltpu.VMEM((f, bt), jnp.int32),
            pltpu.VMEM((bt, d2), jnp.float32),
            pltpu.VMEM((bt, d2), jnp.float32),
            pltpu.VMEM((bt, d), jnp.float32),
            pltpu.VMEM((bt, d), jnp.float32),
            pltpu.SemaphoreType.DMA,
            pltpu.SemaphoreType.DMA,
        ],
        compiler_params=pltpu.CompilerParams(use_tc_tiling_on_sc=True),
    )
    def fk(tpack_hbm, cat_hbm, out_hbm, idx_v, bufa, bufb, s_t, ss_t,
           sema, semb):
        cid = lax.axis_index("c")
        sid = lax.axis_index("s")
        wid = sid * 2 + cid
        bbase = wid * bt

        pltpu.sync_copy(cat_hbm.at[:, pl.ds(bbase, bt)], idx_v)

        zero = jnp.zeros((_LANES,), jnp.float32)

        @pl.loop(0, bt)
        def _(r):
            for i in range(d // _LANES):
                s_t[r, pl.ds(i * _LANES, _LANES)] = zero
                ss_t[r, pl.ds(i * _LANES, _LANES)] = zero

        bufs = (bufa, bufb)
        sems = (sema, semb)

        def gather_q(q, buf, sem):
            return pltpu.async_copy(
                tpack_hbm.at[q // 2].at[idx_v.at[q]], buf, sem)

        @pl.when(True)
        def _():
            gather_q(0, bufa, sema)

        @pl.loop(0, f)
        def _(q):
            parity = lax.rem(q, 2)
            off = lax.rem(q, 2) * d
            for p in range(2):
                @pl.when(parity == p)
                def _():
                    pltpu.make_async_copy(
                        tpack_hbm.at[q // 2].at[idx_v.at[q]],
                        bufs[p], sems[p]).wait()

                    @pl.when(q + 1 < f)
                    def _():
                        gather_q(q + 1, bufs[1 - p], sems[1 - p])

                    buf = bufs[p]

                    @pl.loop(0, bt)
                    def _(r):
                        for c in range(d // _LANES):
                            x = buf[r, pl.ds(off + c * _LANES, _LANES)]
                            plsc.addupdate(
                                s_t.at[r, pl.ds(c * _LANES, _LANES)], x)
                            plsc.addupdate(
                                ss_t.at[r, pl.ds(c * _LANES, _LANES)],
                                x * x)

        pltpu.sync_copy(s_t, out_hbm.at[0, pl.ds(bbase, bt), :])
        pltpu.sync_copy(ss_t, out_hbm.at[1, pl.ds(bbase, bt), :])

    return fk(tpack, cat_t)


def _tc_finish(partials, numeric_t, cat_t, w_num_t, w_cat_t, lin_b2):
    _, n_b, d = partials.shape
    n_num = numeric_t.shape[0]
    f = cat_t.shape[0]
    bb = 512
    g = n_b // bb

    def body(p_ref, num_ref, cat_ref, wn_ref, wc_ref, b_ref, o_ref):
        p = p_ref[...]
        s = p[0]
        ss = p[1]
        order2 = 0.5 * jnp.sum(s * s - ss, axis=1)
        catf = cat_ref[...].astype(jnp.float32)
        order1 = (
            jnp.sum(num_ref[...] * wn_ref[...], axis=0)
            + jnp.sum(catf * wc_ref[...], axis=0)
            + b_ref[0, 0]
        )
        o_ref[...] = jax.nn.sigmoid(order1 + order2).reshape(1, 1, bb)

    return pl.pallas_call(
        body,
        grid=(g,),
        in_specs=[
            pl.BlockSpec((2, bb, d), lambda i: (0, i, 0)),
            pl.BlockSpec((n_num, bb), lambda i: (0, i)),
            pl.BlockSpec((f, bb), lambda i: (0, i)),
            pl.BlockSpec((n_num, 1), lambda i: (0, 0)),
            pl.BlockSpec((f, 1), lambda i: (0, 0)),
            pl.BlockSpec((1, 1), lambda i: (0, 0)),
        ],
        out_specs=pl.BlockSpec((1, 1, bb), lambda i: (i, 0, 0)),
        out_shape=jax.ShapeDtypeStruct((g, 1, bb), jnp.float32),
    )(partials, numeric_t, cat_t, w_num_t, w_cat_t, lin_b2)


def kernel(numeric_features, cat_features, lin_w, lin_b, emb_tables):
    b, f = cat_features.shape
    n_num = numeric_features.shape[1]
    cat_i = cat_features.astype(jnp.int32)
    table_t = jnp.transpose(emb_tables, (0, 2, 1))
    cat_t = jnp.transpose(cat_i, (1, 0))
    numeric_t = jnp.transpose(numeric_features, (1, 0))
    tpack = _tc_pack_transpose(table_t)
    partials = _sc_fm_gather(tpack, cat_t)
    out = _tc_finish(
        partials,
        numeric_t,
        cat_t,
        lin_w[:, :n_num].reshape(n_num, 1),
        lin_w[:, n_num:].reshape(f, 1),
        lin_b.reshape(1, 1),
    )
    return out.reshape(b)


# speedup vs baseline: 3.2029x; 1.4807x over previous
import functools

import jax
import jax.numpy as jnp
import numpy as np
from jax import lax
from jax.experimental import pallas as pl
from jax.experimental.pallas import tpu as pltpu
from jax.experimental.pallas import tpu_sc as plsc

_LANES = 16
_NWORKERS = 32


def _tc_pack_transpose(table_t):
    f, d, v = table_t.shape
    ng = f // 2
    vblk = 5120
    e0_np = np.concatenate([np.eye(d), np.zeros((d, d))], axis=1)
    e1_np = np.concatenate([np.zeros((d, d)), np.eye(d)], axis=1)

    def body(x_ref, e0_ref, e1_ref, o_ref):
        x0 = x_ref[0].astype(jnp.bfloat16)
        x1 = x_ref[1].astype(jnp.bfloat16)
        dn = (((0,), (0,)), ((), ()))
        y = (lax.dot_general(x0, e0_ref[...], dn,
                             preferred_element_type=jnp.float32)
             + lax.dot_general(x1, e1_ref[...], dn,
                               preferred_element_type=jnp.float32))
        o_ref[0] = pltpu.bitcast(y.astype(jnp.bfloat16), jnp.int32)

    return pl.pallas_call(
        body,
        grid=(ng, pl.cdiv(v, vblk)),
        in_specs=[
            pl.BlockSpec((2, d, vblk), lambda g, j: (g, 0, j)),
            pl.BlockSpec((d, 2 * d), lambda g, j: (0, 0)),
            pl.BlockSpec((d, 2 * d), lambda g, j: (0, 0)),
        ],
        out_specs=pl.BlockSpec((1, vblk // 2, 2 * d), lambda g, j: (g, j, 0)),
        out_shape=jax.ShapeDtypeStruct((ng, v // 2, 2 * d), jnp.int32),
    )(table_t,
      jnp.asarray(e0_np, dtype=jnp.bfloat16),
      jnp.asarray(e1_np, dtype=jnp.bfloat16))


def _sc_fm_gather(tpack, cat_t):
    ng, vh, d2 = tpack.shape
    d = d2 // 2
    f = 2 * ng
    n_b = cat_t.shape[1]
    bt = n_b // _NWORKERS

    mesh = plsc.VectorSubcoreMesh(core_axis_name="c", subcore_axis_name="s")

    @functools.partial(
        pl.kernel,
        out_type=jax.ShapeDtypeStruct((2, n_b, d), jnp.float32),
        mesh=mesh,
        scratch_types=[
            pltpu.VMEM((f, bt), jnp.int32),
            pltpu.VMEM((f, bt), jnp.int32),
            pltpu.VMEM((2, bt, d2), jnp.int32),
            pltpu.VMEM((bt, d), jnp.float32),
            pltpu.VMEM((bt, d), jnp.float32),
            pltpu.SemaphoreType.DMA,
            pltpu.SemaphoreType.DMA,
        ],
        compiler_params=pltpu.CompilerParams(use_tc_tiling_on_sc=True,
                                             needs_layout_passes=False),
    )
    def fk(tpack_hbm, cat_hbm, out_hbm, idx_v, pair_v, gbuf, s_t, ss_t,
           sema, semb):
        cid = lax.axis_index("c")
        sid = lax.axis_index("s")
        wid = sid * 2 + cid
        bbase = wid * bt

        pltpu.sync_copy(cat_hbm.at[:, pl.ds(bbase, bt)], idx_v)

        zero = jnp.zeros((_LANES,), jnp.float32)

        @pl.loop(0, f)
        def _(q):
            for i in range(bt // _LANES):
                sl = pl.ds(i * _LANES, _LANES)
                pair_v[q, sl] = lax.shift_right_logical(idx_v[q, sl], 1)

        @pl.loop(0, bt)
        def _(r):
            for i in range(d // _LANES):
                sl = pl.ds(i * _LANES, _LANES)
                s_t[r, sl] = zero
                ss_t[r, sl] = zero

        sems = (sema, semb)

        def gather_q(q, p):
            return pltpu.async_copy(
                tpack_hbm.at[q // 2].at[pair_v.at[q]], gbuf.at[p], sems[p])

        @pl.when(True)
        def _():
            gather_q(0, 0)

        himask = jnp.full((_LANES,), -65536, jnp.int32)

        @pl.loop(0, f)
        def _(q):
            parity = lax.rem(q, 2)
            for p in range(2):
                @pl.when(parity == p)
                def _():
                    pltpu.make_async_copy(
                        tpack_hbm.at[q // 2].at[pair_v.at[q]],
                        gbuf.at[p], sems[p]).wait()

                    @pl.when(q + 1 < f)
                    def _():
                        gather_q(q + 1, 1 - p)

                    buf = gbuf.at[p]

                    @pl.loop(0, bt)
                    def _(r):
                        xv = plsc.load_gather(
                            idx_v, [jnp.full((_LANES,), q, jnp.int32),
                                    jnp.full((_LANES,), r, jnp.int32)])
                        odd = lax.rem(xv, 2) == 1
                        for c in range(d // _LANES):
                            w = buf[r, pl.ds(p * d + c * _LANES, _LANES)]
                            lo = plsc.bitcast(lax.shift_left(w, 16),
                                              jnp.float32)
                            hi = plsc.bitcast(lax.bitwise_and(w, himask),
                                              jnp.float32)
                            x = jnp.where(odd, lo, hi)
                            sl = pl.ds(c * _LANES, _LANES)
                            plsc.addupdate(s_t.at[r, sl], x)
                            plsc.addupdate(ss_t.at[r, sl], x * x)

        pltpu.sync_copy(s_t, out_hbm.at[0, pl.ds(bbase, bt), :])
        pltpu.sync_copy(ss_t, out_hbm.at[1, pl.ds(bbase, bt), :])

    return fk(tpack, cat_t)


def _tc_finish(partials, numeric_t, cat_t, w_num_t, w_cat_t, lin_b2):
    _, n_b, d = partials.shape
    n_num = numeric_t.shape[0]
    f = cat_t.shape[0]
    bb = 512
    g = n_b // bb

    def body(p_ref, num_ref, cat_ref, wn_ref, wc_ref, b_ref, o_ref):
        p = p_ref[...]
        s = p[0]
        ss = p[1]
        order2 = 0.5 * jnp.sum(s * s - ss, axis=1)
        catf = cat_ref[...].astype(jnp.float32)
        order1 = (
            jnp.sum(num_ref[...] * wn_ref[...], axis=0)
            + jnp.sum(catf * wc_ref[...], axis=0)
            + b_ref[0, 0]
        )
        o_ref[...] = jax.nn.sigmoid(order1 + order2).reshape(1, 1, bb)

    return pl.pallas_call(
        body,
        grid=(g,),
        in_specs=[
            pl.BlockSpec((2, bb, d), lambda i: (0, i, 0)),
            pl.BlockSpec((n_num, bb), lambda i: (0, i)),
            pl.BlockSpec((f, bb), lambda i: (0, i)),
            pl.BlockSpec((n_num, 1), lambda i: (0, 0)),
            pl.BlockSpec((f, 1), lambda i: (0, 0)),
            pl.BlockSpec((1, 1), lambda i: (0, 0)),
        ],
        out_specs=pl.BlockSpec((1, 1, bb), lambda i: (i, 0, 0)),
        out_shape=jax.ShapeDtypeStruct((g, 1, bb), jnp.float32),
    )(partials, numeric_t, cat_t, w_num_t, w_cat_t, lin_b2)


def kernel(numeric_features, cat_features, lin_w, lin_b, emb_tables):
    b, f = cat_features.shape
    n_num = numeric_features.shape[1]
    cat_i = cat_features.astype(jnp.int32)
    table_t = jnp.transpose(emb_tables, (0, 2, 1))
    cat_t = jnp.transpose(cat_i, (1, 0))
    numeric_t = jnp.transpose(numeric_features, (1, 0))
    tpack = _tc_pack_transpose(table_t)
    partials = _sc_fm_gather(tpack, cat_t)
    out = _tc_finish(
        partials,
        numeric_t,
        cat_t,
        lin_w[:, :n_num].reshape(n_num, 1),
        lin_w[:, n_num:].reshape(f, 1),
        lin_b.reshape(1, 1),
    )
    return out.reshape(b)
